# baseline (speedup 1.0000x reference)
import functools

import jax
import jax.numpy as jnp
from jax import lax
from jax.experimental import pallas as pl
from jax.experimental.pallas import tpu as pltpu
from jax.experimental.pallas import tpu_sc as plsc

_N = 10000
_E = 320000
_D = 128
_G = 64
_C = 16

_NC = 2
_NS = 16
_NW = _NC * _NS
_K = 80
_GS = 4
_CHF = 192
_CHS = 64
_FAST_CID = 1
_EF = _NS * _CHF * _K
_ES_CAP = _NS * _CHS * _K
_NP = 10240
_RPT = _NP // _NS
_DUMMY = _N


def _sc_agg_body(with_deg, *refs):
    if with_deg:
        (h_hbm, srcr, dstr, z2, z1, acc_out, deg_out,
         sg0, sg1, dg0, dg1, b0, b1, b2, b3, onesv, accs, degs,
         sem0, sem1, sem2, sem3, semg0, semg1) = refs
    else:
        (h_hbm, srcr, dstr, z2, acc_out,
         sg0, sg1, dg0, dg1, b0, b1, b2, b3, accs,
         sem0, sem1, sem2, sem3, semg0, semg1) = refs

    cid = lax.axis_index("c")
    sid = lax.axis_index("s")
    wid = sid * _NC + cid

    pltpu.sync_copy(z2.at[pl.ds(sid * _RPT, _RPT)],
                    accs.at[pl.ds(sid * _RPT, _RPT)])
    if with_deg:
        pltpu.sync_copy(z1.at[pl.ds(sid * _RPT, _RPT)],
                        degs.at[pl.ds(sid * _RPT, _RPT)])
        for t in range(_K // 16):
            onesv[pl.ds(t * 16, 16)] = jnp.ones((16,), jnp.float32)

    pltpu.sync_copy(srcr.at[wid, pl.ds(0, _GS)], sg0)
    pltpu.sync_copy(dstr.at[wid, pl.ds(0, _GS)], dg0)
    pltpu.async_copy(srcr.at[wid, pl.ds(_GS, _GS)], sg1, semg1)
    pltpu.async_copy(dstr.at[wid, pl.ds(_GS, _GS)], dg1, semg1)

    bufs = (b0, b1, b2, b3)
    sems = (sem0, sem1, sem2, sem3)
    for k in range(_GS):
        pltpu.async_copy(h_hbm.at[sg0.at[k]], bufs[k], sems[k])

    plsc.subcore_barrier()

    def scat(buf, dg, k):
        pltpu.sync_copy(buf, accs.at[dg.at[k]], add=True)
        if with_deg:
            pltpu.sync_copy(onesv, degs.at[dg.at[k]], add=True)

    def wait_group(sg, dg, sem):
        pltpu.make_async_copy(srcr.at[wid, pl.ds(0, _GS)], sg, sem).wait()
        pltpu.make_async_copy(dstr.at[wid, pl.ds(0, _GS)], dg, sem).wait()

    def wait_gather(k):
        pltpu.make_async_copy(h_hbm.at[sg0.at[0]], bufs[k], sems[k]).wait()

    steps = jnp.where(cid == _FAST_CID, _CHF // (2 * _GS), _CHS // (2 * _GS))

    def step(j, carry):
        wait_gather(0)
        scat(b0, dg0, 0)
        wait_group(sg1, dg1, semg1)
        pltpu.async_copy(h_hbm.at[sg1.at[0]], b0, sem0)
        wait_gather(1)
        scat(b1, dg0, 1)
        pltpu.async_copy(h_hbm.at[sg1.at[1]], b1, sem1)
        wait_gather(2)
        scat(b2, dg0, 2)
        pltpu.async_copy(h_hbm.at[sg1.at[2]], b2, sem2)
        wait_gather(3)
        scat(b3, dg0, 3)
        pltpu.async_copy(h_hbm.at[sg1.at[3]], b3, sem3)
        g2 = jnp.where(j < steps - 1, 2 * j + 2, 0)
        pltpu.async_copy(srcr.at[wid, pl.ds(g2 * _GS, _GS)], sg0, semg0)
        pltpu.async_copy(dstr.at[wid, pl.ds(g2 * _GS, _GS)], dg0, semg0)
        wait_gather(0)
        scat(b0, dg1, 0)
        wait_group(sg0, dg0, semg0)
        pltpu.async_copy(h_hbm.at[sg0.at[0]], b0, sem0)
        wait_gather(1)
        scat(b1, dg1, 1)
        pltpu.async_copy(h_hbm.at[sg0.at[1]], b1, sem1)
        wait_gather(2)
        scat(b2, dg1, 2)
        pltpu.async_copy(h_hbm.at[sg0.at[2]], b2, sem2)
        wait_gather(3)
        scat(b3, dg1, 3)
        pltpu.async_copy(h_hbm.at[sg0.at[3]], b3, sem3)
        g3 = jnp.where(j < steps - 1, 2 * j + 3, 1)
        pltpu.async_copy(srcr.at[wid, pl.ds(g3 * _GS, _GS)], sg1, semg1)
        pltpu.async_copy(dstr.at[wid, pl.ds(g3 * _GS, _GS)], dg1, semg1)
        return carry

    lax.fori_loop(0, steps, step, 0)
    for k in range(_GS):
        wait_gather(k)
    wait_group(sg1, dg1, semg1)

    plsc.subcore_barrier()

    pltpu.sync_copy(accs.at[pl.ds(sid * _RPT, _RPT)],
                    acc_out.at[cid, pl.ds(sid * _RPT, _RPT)])
    if with_deg:
        @pl.when(sid == 0)
        def _():
            pltpu.sync_copy(degs, deg_out.at[cid])


@functools.cache
def _make_sc_agg(with_deg):
    mesh = plsc.VectorSubcoreMesh(core_axis_name="c", subcore_axis_name="s")
    out_type = [jax.ShapeDtypeStruct((_NC, _NP, _D), jnp.float32)]
    scratch = [
        pltpu.VMEM((_GS, _K), jnp.int32),
        pltpu.VMEM((_GS, _K), jnp.int32),
        pltpu.VMEM((_GS, _K), jnp.int32),
        pltpu.VMEM((_GS, _K), jnp.int32),
        pltpu.VMEM((_K, _D), jnp.float32),
        pltpu.VMEM((_K, _D), jnp.float32),
        pltpu.VMEM((_K, _D), jnp.float32),
        pltpu.VMEM((_K, _D), jnp.float32),
    ]
    if with_deg:
        out_type.append(jax.ShapeDtypeStruct((_NC, _NP), jnp.float32))
        scratch.append(pltpu.VMEM((_K,), jnp.float32))
    scratch.append(pltpu.VMEM_SHARED((_NP, _D), jnp.float32))
    if with_deg:
        scratch.append(pltpu.VMEM_SHARED((_NP,), jnp.float32))
    scratch += [pltpu.SemaphoreType.DMA] * 6
    return pl.kernel(
        functools.partial(_sc_agg_body, with_deg),
        out_type=tuple(out_type),
        mesh=mesh,
        scratch_types=tuple(scratch),
    )


_RB = 2048
_GRID = _NP // _RB


def _combine1_body(accp, degt, h, wlt, wrt, b, out_h, out_inv):
    d = degt[:, 0:1] + degt[:, 1:2]
    inv = 1.0 / jnp.maximum(d, 1.0)
    agg = (accp[0] + accp[1]) * inv
    out_h[...] = jnp.maximum(
        jnp.dot(agg, wlt[...], preferred_element_type=jnp.float32)
        + jnp.dot(h[...], wrt[...], preferred_element_type=jnp.float32)
        + b[...], 0.0)
    out_inv[...] = inv


def _combine23_body(accp, invr, h, wlt, wrt, b, out_h):
    agg = (accp[0] + accp[1]) * invr[...]
    out_h[...] = jnp.maximum(
        jnp.dot(agg, wlt[...], preferred_element_type=jnp.float32)
        + jnp.dot(h[...], wrt[...], preferred_element_type=jnp.float32)
        + b[...], 0.0)


_tc_combine1 = pl.pallas_call(
    _combine1_body,
    grid=(_GRID,),
    in_specs=[
        pl.BlockSpec((_NC, _RB, _D), lambda i: (0, i, 0)),
        pl.BlockSpec((_RB, _NC), lambda i: (i, 0)),
        pl.BlockSpec((_RB, _D), lambda i: (i, 0)),
        pl.BlockSpec((_D, _D), lambda i: (0, 0)),
        pl.BlockSpec((_D, _D), lambda i: (0, 0)),
        pl.BlockSpec((1, _D), lambda i: (0, 0)),
    ],
    out_specs=[
        pl.BlockSpec((_RB, _D), lambda i: (i, 0)),
        pl.BlockSpec((_RB, 1), lambda i: (i, 0)),
    ],
    out_shape=[
        jax.ShapeDtypeStruct((_NP, _D), jnp.float32),
        jax.ShapeDtypeStruct((_NP, 1), jnp.float32),
    ],
)

_tc_combine23 = pl.pallas_call(
    _combine23_body,
    grid=(_GRID,),
    in_specs=[
        pl.BlockSpec((_NC, _RB, _D), lambda i: (0, i, 0)),
        pl.BlockSpec((_RB, 1), lambda i: (i, 0)),
        pl.BlockSpec((_RB, _D), lambda i: (i, 0)),
        pl.BlockSpec((_D, _D), lambda i: (0, 0)),
        pl.BlockSpec((_D, _D), lambda i: (0, 0)),
        pl.BlockSpec((1, _D), lambda i: (0, 0)),
    ],
    out_specs=pl.BlockSpec((_RB, _D), lambda i: (i, 0)),
    out_shape=jax.ShapeDtypeStruct((_NP, _D), jnp.float32),
)


def _pool_body(h3, batchr, wg, bg, w1t, b1, w2t, b2, out):
    h = h3[...]
    gate = jnp.sum(h * wg[...], axis=1, keepdims=True) + bg[...]
    gid = lax.broadcasted_iota(jnp.int32, (1, _G), 1)
    mask = batchr[...] == gid
    gate_eff = jnp.where(mask, gate, -1e30)
    gmax = jnp.max(gate_eff, axis=0, keepdims=True)
    e = jnp.where(mask, jnp.exp(gate_eff - gmax), 0.0)
    denom = jnp.sum(e, axis=0, keepdims=True)
    alpha = e / (denom + 1e-16)
    pooled = lax.dot_general(alpha, h, (((0,), (0,)), ((), ())),
                             preferred_element_type=jnp.float32)
    z = jnp.maximum(
        jnp.dot(pooled, w1t[...], preferred_element_type=jnp.float32)
        + b1[...], 0.0)
    o = (jnp.dot(z, w2t[...], preferred_element_type=jnp.float32)
         + b2[...])
    m = jnp.max(o, axis=1, keepdims=True)
    om = o - m
    out[...] = om - jnp.log(jnp.sum(jnp.exp(om), axis=1, keepdims=True))


_tc_pool = pl.pallas_call(
    _pool_body,
    out_shape=jax.ShapeDtypeStruct((_G, _C), jnp.float32),
)


def kernel(x, edge_index, batch, W1l, b1l, W1r, W2l, b2l, W2r,
           W3l, b3l, W3r, Wg, bg, Wlin1, blin1, Wlin2, blin2):
    xp = jnp.pad(x, ((0, _NP - _N), (0, 0)))
    def _split(a, fill):
        af = a[:_EF].reshape(_NS, _CHF, _K)
        asl = jnp.pad(a[_EF:], (0, _ES_CAP - (_E - _EF)),
                      constant_values=fill).reshape(_NS, _CHS, _K)
        asl = jnp.pad(asl, ((0, 0), (0, _CHF - _CHS), (0, 0)),
                      constant_values=fill)
        parts = (af, asl) if _FAST_CID == 0 else (asl, af)
        return jnp.stack(parts, axis=1).reshape(_NW, _CHF, _K)

    srcp = _split(edge_index[0], 0)
    dstp = _split(edge_index[1], _DUMMY)
    z2 = jnp.zeros((_NP, _D), jnp.float32)
    z1 = jnp.zeros((_NP,), jnp.float32)
    batchp = jnp.pad(batch, (0, _NP - _N), constant_values=_G).reshape(_NP, 1)

    accP, degP = _make_sc_agg(True)(xp, srcp, dstp, z2, z1)
    h1, inv = _tc_combine1(accP, degP.T, xp, W1l.T, W1r.T,
                           b1l.reshape(1, _D))
    accP2, = _make_sc_agg(False)(h1, srcp, dstp, z2)
    h2 = _tc_combine23(accP2, inv, h1, W2l.T, W2r.T, b2l.reshape(1, _D))
    accP3, = _make_sc_agg(False)(h2, srcp, dstp, z2)
    h3 = _tc_combine23(accP3, inv, h2, W3l.T, W3r.T, b3l.reshape(1, _D))
    out = _tc_pool(h3, batchp, Wg, bg.reshape(1, 1), Wlin1.T,
                   blin1.reshape(1, _D), Wlin2.T, blin2.reshape(1, _C))
    return out


# speedup vs baseline: 1.0827x; 1.0827x over previous
import functools

import jax
import jax.numpy as jnp
from jax import lax
from jax.experimental import pallas as pl
from jax.experimental.pallas import tpu as pltpu
from jax.experimental.pallas import tpu_sc as plsc

_N = 10000
_E = 320000
_D = 128
_G = 64
_C = 16

_NC = 2
_NS = 16
_NW = _NC * _NS
_K = 128
_CH = 80
_GS = 4
_NG = _CH // _GS
_EPT = _K * _CH
_EPAD = _NW * _EPT
_NP = 10240
_RPT = _NP // _NS
_DUMMY = _N


def _sc_agg_body(with_deg, *refs):
    if with_deg:
        (h_hbm, srcr, dstr, z2, z1, acc_out, deg_out,
         sg0, sg1, dg0, dg1, b0, b1, onesv, accs, degs,
         sem0, sem1, semg0, semg1) = refs
    else:
        (h_hbm, srcr, dstr, z2, acc_out,
         sg0, sg1, dg0, dg1, b0, b1, accs,
         sem0, sem1, semg0, semg1) = refs

    cid = lax.axis_index("c")
    sid = lax.axis_index("s")
    wid = sid * _NC + cid

    pltpu.sync_copy(z2.at[pl.ds(sid * _RPT, _RPT)],
                    accs.at[pl.ds(sid * _RPT, _RPT)])
    if with_deg:
        pltpu.sync_copy(z1.at[pl.ds(sid * _RPT, _RPT)],
                        degs.at[pl.ds(sid * _RPT, _RPT)])
        for t in range(_K // 16):
            onesv[pl.ds(t * 16, 16)] = jnp.ones((16,), jnp.float32)

    pltpu.sync_copy(srcr.at[wid, pl.ds(0, _GS)], sg0)
    pltpu.sync_copy(dstr.at[wid, pl.ds(0, _GS)], dg0)
    pltpu.async_copy(srcr.at[wid, pl.ds(_GS, _GS)], sg1, semg1)
    pltpu.async_copy(dstr.at[wid, pl.ds(_GS, _GS)], dg1, semg1)

    pltpu.async_copy(h_hbm.at[sg0.at[0]], b0, sem0)

    plsc.subcore_barrier()

    def scat(buf, dg, k):
        pltpu.sync_copy(buf, accs.at[dg.at[k]], add=True)
        if with_deg:
            pltpu.sync_copy(onesv, degs.at[dg.at[k]], add=True)

    def wait_group(sg, dg, sem):
        pltpu.make_async_copy(srcr.at[wid, pl.ds(0, _GS)], sg, sem).wait()
        pltpu.make_async_copy(dstr.at[wid, pl.ds(0, _GS)], dg, sem).wait()

    def step(j, carry):
        pltpu.async_copy(h_hbm.at[sg0.at[1]], b1, sem1)
        pltpu.make_async_copy(h_hbm.at[sg0.at[0]], b0, sem0).wait()
        scat(b0, dg0, 0)
        pltpu.async_copy(h_hbm.at[sg0.at[2]], b0, sem0)
        pltpu.make_async_copy(h_hbm.at[sg0.at[1]], b1, sem1).wait()
        scat(b1, dg0, 1)
        pltpu.async_copy(h_hbm.at[sg0.at[3]], b1, sem1)
        pltpu.make_async_copy(h_hbm.at[sg0.at[2]], b0, sem0).wait()
        scat(b0, dg0, 2)
        wait_group(sg1, dg1, semg1)
        pltpu.async_copy(h_hbm.at[sg1.at[0]], b0, sem0)
        pltpu.make_async_copy(h_hbm.at[sg0.at[3]], b1, sem1).wait()
        scat(b1, dg0, 3)
        g2 = jnp.where(j < _NG // 2 - 1, 2 * j + 2, 0)
        pltpu.async_copy(srcr.at[wid, pl.ds(g2 * _GS, _GS)], sg0, semg0)
        pltpu.async_copy(dstr.at[wid, pl.ds(g2 * _GS, _GS)], dg0, semg0)
        pltpu.async_copy(h_hbm.at[sg1.at[1]], b1, sem1)
        pltpu.make_async_copy(h_hbm.at[sg1.at[0]], b0, sem0).wait()
        scat(b0, dg1, 0)
        pltpu.async_copy(h_hbm.at[sg1.at[2]], b0, sem0)
        pltpu.make_async_copy(h_hbm.at[sg1.at[1]], b1, sem1).wait()
        scat(b1, dg1, 1)
        pltpu.async_copy(h_hbm.at[sg1.at[3]], b1, sem1)
        pltpu.make_async_copy(h_hbm.at[sg1.at[2]], b0, sem0).wait()
        scat(b0, dg1, 2)
        wait_group(sg0, dg0, semg0)
        pltpu.async_copy(h_hbm.at[sg0.at[0]], b0, sem0)
        pltpu.make_async_copy(h_hbm.at[sg1.at[3]], b1, sem1).wait()
        scat(b1, dg1, 3)
        g3 = jnp.where(j < _NG // 2 - 1, 2 * j + 3, 1)
        pltpu.async_copy(srcr.at[wid, pl.ds(g3 * _GS, _GS)], sg1, semg1)
        pltpu.async_copy(dstr.at[wid, pl.ds(g3 * _GS, _GS)], dg1, semg1)
        return carry

    lax.fori_loop(0, _NG // 2, step, 0)
    pltpu.make_async_copy(h_hbm.at[sg0.at[0]], b0, sem0).wait()
    wait_group(sg1, dg1, semg1)

    plsc.subcore_barrier()

    pltpu.sync_copy(accs.at[pl.ds(sid * _RPT, _RPT)],
                    acc_out.at[cid, pl.ds(sid * _RPT, _RPT)])
    if with_deg:
        @pl.when(sid == 0)
        def _():
            pltpu.sync_copy(degs, deg_out.at[cid])


@functools.cache
def _make_sc_agg(with_deg):
    mesh = plsc.VectorSubcoreMesh(core_axis_name="c", subcore_axis_name="s")
    out_type = [jax.ShapeDtypeStruct((_NC, _NP, _D), jnp.float32)]
    scratch = [
        pltpu.VMEM((_GS, _K), jnp.int32),
        pltpu.VMEM((_GS, _K), jnp.int32),
        pltpu.VMEM((_GS, _K), jnp.int32),
        pltpu.VMEM((_GS, _K), jnp.int32),
        pltpu.VMEM((_K, _D), jnp.float32),
        pltpu.VMEM((_K, _D), jnp.float32),
    ]
    if with_deg:
        out_type.append(jax.ShapeDtypeStruct((_NC, _NP), jnp.float32))
        scratch.append(pltpu.VMEM((_K,), jnp.float32))
    scratch.append(pltpu.VMEM_SHARED((_NP, _D), jnp.float32))
    if with_deg:
        scratch.append(pltpu.VMEM_SHARED((_NP,), jnp.float32))
    scratch += [pltpu.SemaphoreType.DMA] * 4
    return pl.kernel(
        functools.partial(_sc_agg_body, with_deg),
        out_type=tuple(out_type),
        mesh=mesh,
        scratch_types=tuple(scratch),
    )


_RB = 2048
_GRID = _NP // _RB


def _combine1_body(accp, degt, h, wlt, wrt, b, out_h, out_inv):
    d = degt[:, 0:1] + degt[:, 1:2]
    inv = 1.0 / jnp.maximum(d, 1.0)
    agg = (accp[0] + accp[1]) * inv
    out_h[...] = jnp.maximum(
        jnp.dot(agg, wlt[...], preferred_element_type=jnp.float32)
        + jnp.dot(h[...], wrt[...], preferred_element_type=jnp.float32)
        + b[...], 0.0)
    out_inv[...] = inv


def _combine23_body(accp, invr, h, wlt, wrt, b, out_h):
    agg = (accp[0] + accp[1]) * invr[...]
    out_h[...] = jnp.maximum(
        jnp.dot(agg, wlt[...], preferred_element_type=jnp.float32)
        + jnp.dot(h[...], wrt[...], preferred_element_type=jnp.float32)
        + b[...], 0.0)


_tc_combine1 = pl.pallas_call(
    _combine1_body,
    grid=(_GRID,),
    in_specs=[
        pl.BlockSpec((_NC, _RB, _D), lambda i: (0, i, 0)),
        pl.BlockSpec((_RB, _NC), lambda i: (i, 0)),
        pl.BlockSpec((_RB, _D), lambda i: (i, 0)),
        pl.BlockSpec((_D, _D), lambda i: (0, 0)),
        pl.BlockSpec((_D, _D), lambda i: (0, 0)),
        pl.BlockSpec((1, _D), lambda i: (0, 0)),
    ],
    out_specs=[
        pl.BlockSpec((_RB, _D), lambda i: (i, 0)),
        pl.BlockSpec((_RB, 1), lambda i: (i, 0)),
    ],
    out_shape=[
        jax.ShapeDtypeStruct((_NP, _D), jnp.float32),
        jax.ShapeDtypeStruct((_NP, 1), jnp.float32),
    ],
)

_tc_combine23 = pl.pallas_call(
    _combine23_body,
    grid=(_GRID,),
    in_specs=[
        pl.BlockSpec((_NC, _RB, _D), lambda i: (0, i, 0)),
        pl.BlockSpec((_RB, 1), lambda i: (i, 0)),
        pl.BlockSpec((_RB, _D), lambda i: (i, 0)),
        pl.BlockSpec((_D, _D), lambda i: (0, 0)),
        pl.BlockSpec((_D, _D), lambda i: (0, 0)),
        pl.BlockSpec((1, _D), lambda i: (0, 0)),
    ],
    out_specs=pl.BlockSpec((_RB, _D), lambda i: (i, 0)),
    out_shape=jax.ShapeDtypeStruct((_NP, _D), jnp.float32),
)


def _pool_body(h3, batchr, wg, bg, w1t, b1, w2t, b2, out):
    h = h3[...]
    gate = jnp.sum(h * wg[...], axis=1, keepdims=True) + bg[...]
    gid = lax.broadcasted_iota(jnp.int32, (1, _G), 1)
    mask = batchr[...] == gid
    gate_eff = jnp.where(mask, gate, -1e30)
    gmax = jnp.max(gate_eff, axis=0, keepdims=True)
    e = jnp.where(mask, jnp.exp(gate_eff - gmax), 0.0)
    denom = jnp.sum(e, axis=0, keepdims=True)
    alpha = e / (denom + 1e-16)
    pooled = lax.dot_general(alpha, h, (((0,), (0,)), ((), ())),
                             preferred_element_type=jnp.float32)
    z = jnp.maximum(
        jnp.dot(pooled, w1t[...], preferred_element_type=jnp.float32)
        + b1[...], 0.0)
    o = (jnp.dot(z, w2t[...], preferred_element_type=jnp.float32)
         + b2[...])
    m = jnp.max(o, axis=1, keepdims=True)
    om = o - m
    out[...] = om - jnp.log(jnp.sum(jnp.exp(om), axis=1, keepdims=True))


_tc_pool = pl.pallas_call(
    _pool_body,
    out_shape=jax.ShapeDtypeStruct((_G, _C), jnp.float32),
)


def kernel(x, edge_index, batch, W1l, b1l, W1r, W2l, b2l, W2r,
           W3l, b3l, W3r, Wg, bg, Wlin1, blin1, Wlin2, blin2):
    xp = jnp.pad(x, ((0, _NP - _N), (0, 0)))
    srcp = jnp.pad(edge_index[0], (0, _EPAD - _E)).reshape(_NW, _CH, _K)
    dstp = jnp.pad(edge_index[1], (0, _EPAD - _E),
                   constant_values=_DUMMY).reshape(_NW, _CH, _K)
    z2 = jnp.zeros((_NP, _D), jnp.float32)
    z1 = jnp.zeros((_NP,), jnp.float32)
    batchp = jnp.pad(batch, (0, _NP - _N), constant_values=_G).reshape(_NP, 1)

    accP, degP = _make_sc_agg(True)(xp, srcp, dstp, z2, z1)
    h1, inv = _tc_combine1(accP, degP.T, xp, W1l.T, W1r.T,
                           b1l.reshape(1, _D))
    accP2, = _make_sc_agg(False)(h1, srcp, dstp, z2)
    h2 = _tc_combine23(accP2, inv, h1, W2l.T, W2r.T, b2l.reshape(1, _D))
    accP3, = _make_sc_agg(False)(h2, srcp, dstp, z2)
    h3 = _tc_combine23(accP3, inv, h2, W3l.T, W3r.T, b3l.reshape(1, _D))
    out = _tc_pool(h3, batchp, Wg, bg.reshape(1, 1), Wlin1.T,
                   blin1.reshape(1, _D), Wlin2.T, blin2.reshape(1, _C))
    return out


# speedup vs baseline: 1.1023x; 1.0181x over previous
import functools

import jax
import jax.numpy as jnp
from jax import lax
from jax.experimental import pallas as pl
from jax.experimental.pallas import tpu as pltpu
from jax.experimental.pallas import tpu_sc as plsc

_N = 10000
_E = 320000
_D = 128
_G = 64
_C = 16

_NC = 2
_NS = 16
_NW = _NC * _NS
_K = 128
_CH = 80
_GS = 4
_NG = _CH // _GS
_EPT = _K * _CH
_EPAD = _NW * _EPT
_NP = 10240
_RPT = _NP // _NS
_DUMMY = _N


def _sc_agg_body(with_deg, *refs):
    if with_deg:
        (h_hbm, srcr, dstr, z2, z1, acc_out, deg_out,
         sg0, sg1, dg0, dg1, b0, b1, onesv, accs, degs,
         sem0, sem1, semg0, semg1) = refs
    else:
        (h_hbm, srcr, dstr, z2, acc_out,
         sg0, sg1, dg0, dg1, b0, b1, accs,
         sem0, sem1, semg0, semg1) = refs

    cid = lax.axis_index("c")
    sid = lax.axis_index("s")
    wid = sid * _NC + cid

    pltpu.sync_copy(z2.at[pl.ds(sid * _RPT, _RPT)],
                    accs.at[pl.ds(sid * _RPT, _RPT)])
    if with_deg:
        pltpu.sync_copy(z1.at[pl.ds(sid * _RPT, _RPT)],
                        degs.at[pl.ds(sid * _RPT, _RPT)])
        for t in range(_K // 16):
            onesv[pl.ds(t * 16, 16)] = jnp.ones((16,), jnp.float32)

    pltpu.sync_copy(srcr.at[wid, pl.ds(0, _GS)], sg0)
    pltpu.sync_copy(dstr.at[wid, pl.ds(0, _GS)], dg0)
    pltpu.async_copy(srcr.at[wid, pl.ds(_GS, _GS)], sg1, semg1)
    pltpu.async_copy(dstr.at[wid, pl.ds(_GS, _GS)], dg1, semg1)

    pltpu.async_copy(h_hbm.at[sg0.at[0]], b0, sem0)

    plsc.subcore_barrier()

    def scat(buf, dg, k):
        pltpu.sync_copy(buf, accs.at[dg.at[k]], add=True)
        if with_deg:
            pltpu.sync_copy(onesv, degs.at[dg.at[k]], add=True)

    def wait_group(sg, dg, sem):
        pltpu.make_async_copy(srcr.at[wid, pl.ds(0, _GS)], sg, sem).wait()
        pltpu.make_async_copy(dstr.at[wid, pl.ds(0, _GS)], dg, sem).wait()

    def step(j, carry):
        pltpu.async_copy(h_hbm.at[sg0.at[1]], b1, sem1)
        pltpu.make_async_copy(h_hbm.at[sg0.at[0]], b0, sem0).wait()
        scat(b0, dg0, 0)
        pltpu.async_copy(h_hbm.at[sg0.at[2]], b0, sem0)
        pltpu.make_async_copy(h_hbm.at[sg0.at[1]], b1, sem1).wait()
        scat(b1, dg0, 1)
        pltpu.async_copy(h_hbm.at[sg0.at[3]], b1, sem1)
        pltpu.make_async_copy(h_hbm.at[sg0.at[2]], b0, sem0).wait()
        scat(b0, dg0, 2)
        wait_group(sg1, dg1, semg1)
        pltpu.async_copy(h_hbm.at[sg1.at[0]], b0, sem0)
        pltpu.make_async_copy(h_hbm.at[sg0.at[3]], b1, sem1).wait()
        scat(b1, dg0, 3)
        g2 = jnp.where(j < _NG // 2 - 1, 2 * j + 2, 0)
        pltpu.async_copy(srcr.at[wid, pl.ds(g2 * _GS, _GS)], sg0, semg0)
        pltpu.async_copy(dstr.at[wid, pl.ds(g2 * _GS, _GS)], dg0, semg0)
        pltpu.async_copy(h_hbm.at[sg1.at[1]], b1, sem1)
        pltpu.make_async_copy(h_hbm.at[sg1.at[0]], b0, sem0).wait()
        scat(b0, dg1, 0)
        pltpu.async_copy(h_hbm.at[sg1.at[2]], b0, sem0)
        pltpu.make_async_copy(h_hbm.at[sg1.at[1]], b1, sem1).wait()
        scat(b1, dg1, 1)
        pltpu.async_copy(h_hbm.at[sg1.at[3]], b1, sem1)
        pltpu.make_async_copy(h_hbm.at[sg1.at[2]], b0, sem0).wait()
        scat(b0, dg1, 2)
        wait_group(sg0, dg0, semg0)
        pltpu.async_copy(h_hbm.at[sg0.at[0]], b0, sem0)
        pltpu.make_async_copy(h_hbm.at[sg1.at[3]], b1, sem1).wait()
        scat(b1, dg1, 3)
        g3 = jnp.where(j < _NG // 2 - 1, 2 * j + 3, 1)
        pltpu.async_copy(srcr.at[wid, pl.ds(g3 * _GS, _GS)], sg1, semg1)
        pltpu.async_copy(dstr.at[wid, pl.ds(g3 * _GS, _GS)], dg1, semg1)
        return carry

    lax.fori_loop(0, _NG // 2, step, 0)
    pltpu.make_async_copy(h_hbm.at[sg0.at[0]], b0, sem0).wait()
    wait_group(sg1, dg1, semg1)

    plsc.subcore_barrier()

    pltpu.sync_copy(accs.at[pl.ds(sid * _RPT, _RPT)],
                    acc_out.at[cid, pl.ds(sid * _RPT, _RPT)])
    if with_deg:
        @pl.when(sid == 0)
        def _():
            pltpu.sync_copy(degs, deg_out.at[cid])


@functools.cache
def _make_sc_agg(with_deg):
    mesh = plsc.VectorSubcoreMesh(core_axis_name="c", subcore_axis_name="s")
    out_type = [jax.ShapeDtypeStruct((_NC, _NP, _D), jnp.float32)]
    scratch = [
        pltpu.VMEM((_GS, _K), jnp.int32),
        pltpu.VMEM((_GS, _K), jnp.int32),
        pltpu.VMEM((_GS, _K), jnp.int32),
        pltpu.VMEM((_GS, _K), jnp.int32),
        pltpu.VMEM((_K, _D), jnp.float32),
        pltpu.VMEM((_K, _D), jnp.float32),
    ]
    if with_deg:
        out_type.append(jax.ShapeDtypeStruct((_NC, _NP), jnp.float32))
        scratch.append(pltpu.VMEM((_K,), jnp.float32))
    scratch.append(pltpu.VMEM_SHARED((_NP, _D), jnp.float32))
    if with_deg:
        scratch.append(pltpu.VMEM_SHARED((_NP,), jnp.float32))
    scratch += [pltpu.SemaphoreType.DMA] * 4
    return pl.kernel(
        functools.partial(_sc_agg_body, with_deg),
        out_type=tuple(out_type),
        mesh=mesh,
        scratch_types=tuple(scratch),
    )


_RB = 2048
_GRID = _NP // _RB


def _root_body(h, wrt, b, out):
    out[...] = (jnp.dot(h[...], wrt[...], preferred_element_type=jnp.float32)
                + b[...])


_tc_root = pl.pallas_call(
    _root_body,
    grid=(_GRID,),
    in_specs=[
        pl.BlockSpec((_RB, _D), lambda i: (i, 0)),
        pl.BlockSpec((_D, _D), lambda i: (0, 0)),
        pl.BlockSpec((1, _D), lambda i: (0, 0)),
    ],
    out_specs=pl.BlockSpec((_RB, _D), lambda i: (i, 0)),
    out_shape=jax.ShapeDtypeStruct((_NP, _D), jnp.float32),
)


def _combine1_body(accp, degt, hr, wlt, out_h, out_inv):
    d = degt[:, 0:1] + degt[:, 1:2]
    inv = 1.0 / jnp.maximum(d, 1.0)
    agg = (accp[0] + accp[1]) * inv
    out_h[...] = jnp.maximum(
        jnp.dot(agg, wlt[...], preferred_element_type=jnp.float32)
        + hr[...], 0.0)
    out_inv[...] = inv


def _combine23_body(accp, invr, hr, wlt, out_h):
    agg = (accp[0] + accp[1]) * invr[...]
    out_h[...] = jnp.maximum(
        jnp.dot(agg, wlt[...], preferred_element_type=jnp.float32)
        + hr[...], 0.0)


_tc_combine1 = pl.pallas_call(
    _combine1_body,
    grid=(_GRID,),
    in_specs=[
        pl.BlockSpec((_NC, _RB, _D), lambda i: (0, i, 0)),
        pl.BlockSpec((_RB, _NC), lambda i: (i, 0)),
        pl.BlockSpec((_RB, _D), lambda i: (i, 0)),
        pl.BlockSpec((_D, _D), lambda i: (0, 0)),
    ],
    out_specs=[
        pl.BlockSpec((_RB, _D), lambda i: (i, 0)),
        pl.BlockSpec((_RB, 1), lambda i: (i, 0)),
    ],
    out_shape=[
        jax.ShapeDtypeStruct((_NP, _D), jnp.float32),
        jax.ShapeDtypeStruct((_NP, 1), jnp.float32),
    ],
)

_tc_combine23 = pl.pallas_call(
    _combine23_body,
    grid=(_GRID,),
    in_specs=[
        pl.BlockSpec((_NC, _RB, _D), lambda i: (0, i, 0)),
        pl.BlockSpec((_RB, 1), lambda i: (i, 0)),
        pl.BlockSpec((_RB, _D), lambda i: (i, 0)),
        pl.BlockSpec((_D, _D), lambda i: (0, 0)),
    ],
    out_specs=pl.BlockSpec((_RB, _D), lambda i: (i, 0)),
    out_shape=jax.ShapeDtypeStruct((_NP, _D), jnp.float32),
)


def _pool_body(h3, batchr, wg, bg, w1t, b1, w2t, b2, out):
    h = h3[...]
    gate = jnp.sum(h * wg[...], axis=1, keepdims=True) + bg[...]
    gid = lax.broadcasted_iota(jnp.int32, (1, _G), 1)
    mask = batchr[...] == gid
    gate_eff = jnp.where(mask, gate, -1e30)
    gmax = jnp.max(gate_eff, axis=0, keepdims=True)
    e = jnp.where(mask, jnp.exp(gate_eff - gmax), 0.0)
    denom = jnp.sum(e, axis=0, keepdims=True)
    alpha = e / (denom + 1e-16)
    pooled = lax.dot_general(alpha, h, (((0,), (0,)), ((), ())),
                             preferred_element_type=jnp.float32)
    z = jnp.maximum(
        jnp.dot(pooled, w1t[...], preferred_element_type=jnp.float32)
        + b1[...], 0.0)
    o = (jnp.dot(z, w2t[...], preferred_element_type=jnp.float32)
         + b2[...])
    m = jnp.max(o, axis=1, keepdims=True)
    om = o - m
    out[...] = om - jnp.log(jnp.sum(jnp.exp(om), axis=1, keepdims=True))


_tc_pool = pl.pallas_call(
    _pool_body,
    out_shape=jax.ShapeDtypeStruct((_G, _C), jnp.float32),
)


def kernel(x, edge_index, batch, W1l, b1l, W1r, W2l, b2l, W2r,
           W3l, b3l, W3r, Wg, bg, Wlin1, blin1, Wlin2, blin2):
    xp = jnp.pad(x, ((0, _NP - _N), (0, 0)))
    srcp = jnp.pad(edge_index[0], (0, _EPAD - _E)).reshape(_NW, _CH, _K)
    dstp = jnp.pad(edge_index[1], (0, _EPAD - _E),
                   constant_values=_DUMMY).reshape(_NW, _CH, _K)
    z2 = jnp.zeros((_NP, _D), jnp.float32)
    z1 = jnp.zeros((_NP,), jnp.float32)
    batchp = jnp.pad(batch, (0, _NP - _N), constant_values=_G).reshape(_NP, 1)

    hr1 = _tc_root(xp, W1r.T, b1l.reshape(1, _D))
    accP, degP = _make_sc_agg(True)(xp, srcp, dstp, z2, z1)
    h1, inv = _tc_combine1(accP, degP.T, hr1, W1l.T)
    hr2 = _tc_root(h1, W2r.T, b2l.reshape(1, _D))
    accP2, = _make_sc_agg(False)(h1, srcp, dstp, z2)
    h2 = _tc_combine23(accP2, inv, hr2, W2l.T)
    hr3 = _tc_root(h2, W3r.T, b3l.reshape(1, _D))
    accP3, = _make_sc_agg(False)(h2, srcp, dstp, z2)
    h3 = _tc_combine23(accP3, inv, hr3, W3l.T)
    out = _tc_pool(h3, batchp, Wg, bg.reshape(1, 1), Wlin1.T,
                   blin1.reshape(1, _D), Wlin2.T, blin2.reshape(1, _C))
    return out
